# Initial kernel scaffold; baseline (speedup 1.0000x reference)
import jax
import jax.numpy as jnp
from jax.experimental import pallas as pl


def kernel(x, edge_attr, edge_index, u, We, be, Wv, bv, Wg, bg):
    raise NotImplementedError("write your pallas kernel here")


# speedup vs baseline: 4.4159x; 4.4159x over previous
import functools

import jax
import jax.numpy as jnp
from jax import lax
from jax.experimental import pallas as pl
from jax.experimental.pallas import tpu as pltpu
from jax.experimental.pallas import tpu_sc as plsc


def _tc_nodes_body(x_ref, We_ref, xs_ref, xd_ref, *, D, DE):
    f32 = jnp.float32
    xblk = x_ref[...]
    xs_ref[...] = jnp.dot(xblk, We_ref[DE:DE + D, :], preferred_element_type=f32)
    xd_ref[...] = jnp.dot(xblk, We_ref[DE + D:DE + 2 * D, :],
                          preferred_element_type=f32)


def _tc_transpose_body(e_ref, eT_ref, *, DE):
    f32 = jnp.float32
    ident = (jax.lax.broadcasted_iota(jnp.int32, (DE, DE), 0)
             == jax.lax.broadcasted_iota(jnp.int32, (DE, DE), 1)).astype(f32)
    eT_ref[...] = lax.dot_general(ident, e_ref[...],
                                  (((1,), (1,)), ((), ())),
                                  preferred_element_type=f32)


def _tc_edges_body(eaT_ref, We_ref, u_ref, be_ref, t_ref, *, D, DE):
    f32 = jnp.float32
    ce = jnp.dot(u_ref[...], We_ref[DE + 2 * D:, :],
                 preferred_element_type=f32) + be_ref[...]
    t_ref[...] = lax.dot_general(
        eaT_ref[...], We_ref[:DE, :],
        (((0,), (0,)), ((), ())), preferred_element_type=f32) + ce


def _tc_node_body(x_ref, pin_ref, pout_ref, Wv_ref, Wg_ref, u_ref, bv_ref,
                  bg_ref, xn_ref, un_ref, accx_ref, acce_ref,
                  *, D, DE, DU, N, E, NBLK):
    f32 = jnp.float32
    i = pl.program_id(0)

    @pl.when(i == 0)
    def _():
        accx_ref[...] = jnp.zeros_like(accx_ref)
        acce_ref[...] = jnp.zeros_like(acce_ref)

    aggin = pin_ref[0] + pin_ref[1]
    aggout = pout_ref[0] + pout_ref[1]
    cv = jnp.dot(u_ref[...], Wv_ref[D + 2 * DE:, :],
                 preferred_element_type=f32) + bv_ref[...]
    xn = (jnp.dot(x_ref[...], Wv_ref[:D, :], preferred_element_type=f32)
          + jnp.dot(aggin, Wv_ref[D:D + DE, :], preferred_element_type=f32)
          + jnp.dot(aggout, Wv_ref[D + DE:D + 2 * DE, :],
                    preferred_element_type=f32)
          + cv)
    xn = jnp.maximum(xn, 0.0)
    xn_ref[...] = xn
    accx_ref[...] += jnp.sum(xn, axis=0, keepdims=True)
    acce_ref[...] += jnp.sum(aggin, axis=0, keepdims=True)

    @pl.when(i == NBLK - 1)
    def _():
        gx = jnp.dot(accx_ref[...] * (1.0 / N), Wg_ref[:D, :],
                     preferred_element_type=f32)
        ge = jnp.dot(acce_ref[...] * (1.0 / E), Wg_ref[D:D + DE, :],
                     preferred_element_type=f32)
        gu = jnp.dot(u_ref[...], Wg_ref[D + DE:, :],
                     preferred_element_type=f32)
        un_ref[...] = jnp.maximum(gx + ge + gu + bg_ref[...], 0.0)


def _make_sc_edge(N, E, DE, NC, NS, C):
    NW = NC * NS
    EPW = E // NW
    NCH = EPW // C
    NPT = (N // NS) // 8 * 8
    NPT_LAST = N - NPT * (NS - 1)
    mesh = plsc.VectorSubcoreMesh(core_axis_name="c", subcore_axis_name="s")
    f32 = jnp.float32

    @functools.partial(
        pl.kernel,
        out_type=[
            jax.ShapeDtypeStruct((E, DE), f32),
            jax.ShapeDtypeStruct((NC, N, DE), f32),
            jax.ShapeDtypeStruct((NC, N, DE), f32),
        ],
        mesh=mesh,
        scratch_types=[
            pltpu.VMEM((C,), jnp.int32),
            pltpu.VMEM((C,), jnp.int32),
            pltpu.VMEM((C, DE), f32),
            pltpu.VMEM((C, DE), f32),
            pltpu.VMEM((C, DE), f32),
            pltpu.VMEM((C, DE), f32),
            pltpu.VMEM((NPT_LAST, DE), f32),
            pltpu.VMEM_SHARED((N, DE), f32),
            pltpu.VMEM_SHARED((N, DE), f32),
            pltpu.SemaphoreType.DMA,
            pltpu.SemaphoreType.DMA,
        ],
        compiler_params=pltpu.CompilerParams(use_tc_tiling_on_sc=False),
    )
    def sc_edge(xs_hbm, xd_hbm, t_hbm, src_hbm, dst_hbm,
                e_hbm, pin_hbm, pout_hbm,
                srcb, dstb, xsb, xdb, tb, eb, zb, agg_in, agg_out,
                sem0, sem1):
        cid = lax.axis_index("c")
        sid = lax.axis_index("s")
        wid = sid * NC + cid

        zrow = jnp.zeros((16,), f32)

        def zfill(i, _):
            zb[i] = zrow
            return 0

        lax.fori_loop(0, NPT_LAST, zfill, 0, unroll=8)
        nbase = sid * NPT

        @pl.when(sid < NS - 1)
        def _():
            pltpu.sync_copy(zb.at[pl.ds(0, NPT)], agg_in.at[pl.ds(nbase, NPT)])
            pltpu.sync_copy(zb.at[pl.ds(0, NPT)], agg_out.at[pl.ds(nbase, NPT)])

        @pl.when(sid == NS - 1)
        def _():
            pltpu.sync_copy(zb, agg_in.at[pl.ds(nbase, NPT_LAST)])
            pltpu.sync_copy(zb, agg_out.at[pl.ds(nbase, NPT_LAST)])

        plsc.subcore_barrier()

        ebase = wid * EPW

        def chunk(j, _):
            base = ebase + j * C
            pltpu.sync_copy(src_hbm.at[pl.ds(base, C)], srcb)
            pltpu.sync_copy(dst_hbm.at[pl.ds(base, C)], dstb)
            g0 = pltpu.async_copy(xs_hbm.at[srcb], xsb, sem0)
            g1 = pltpu.async_copy(xd_hbm.at[dstb], xdb, sem1)
            pltpu.sync_copy(t_hbm.at[pl.ds(base, C)], tb)
            g0.wait()
            g1.wait()

            def row(i, _):
                eb[i] = jnp.maximum(tb[i] + xsb[i] + xdb[i], 0.0)
                return 0

            lax.fori_loop(0, C, row, 0, unroll=8)
            pltpu.sync_copy(eb, e_hbm.at[pl.ds(base, C)])
            pltpu.sync_copy(eb, agg_in.at[dstb], add=True)
            pltpu.sync_copy(eb, agg_out.at[srcb], add=True)
            return 0

        lax.fori_loop(0, NCH, chunk, 0)

        plsc.subcore_barrier()

        @pl.when(sid < NS - 1)
        def _():
            pltpu.sync_copy(agg_in.at[pl.ds(nbase, NPT)],
                            pin_hbm.at[cid, pl.ds(nbase, NPT)])
            pltpu.sync_copy(agg_out.at[pl.ds(nbase, NPT)],
                            pout_hbm.at[cid, pl.ds(nbase, NPT)])

        @pl.when(sid == NS - 1)
        def _():
            pltpu.sync_copy(agg_in.at[pl.ds(nbase, NPT_LAST)],
                            pin_hbm.at[cid, pl.ds(nbase, NPT_LAST)])
            pltpu.sync_copy(agg_out.at[pl.ds(nbase, NPT_LAST)],
                            pout_hbm.at[cid, pl.ds(nbase, NPT_LAST)])

    return sc_edge


def kernel(x, edge_attr, edge_index, u, We, be, Wv, bv, Wg, bg):
    N, D = x.shape
    E, DE = edge_attr.shape
    DU = u.shape[0]
    f32 = jnp.float32

    src = edge_index[0]
    dst = edge_index[1]
    u2 = u.reshape(1, DU)
    be2 = be.reshape(1, DE)
    bv2 = bv.reshape(1, D)
    bg2 = bg.reshape(1, DU)

    NBLK = 10
    BN = N // NBLK
    BE = 2560
    EBLK = E // BE

    xs, xd = pl.pallas_call(
        functools.partial(_tc_nodes_body, D=D, DE=DE),
        grid=(NBLK,),
        in_specs=[
            pl.BlockSpec((BN, D), lambda i: (i, 0)),
            pl.BlockSpec(We.shape, lambda i: (0, 0)),
        ],
        out_specs=[
            pl.BlockSpec((BN, DE), lambda i: (i, 0)),
            pl.BlockSpec((BN, DE), lambda i: (i, 0)),
        ],
        out_shape=[
            jax.ShapeDtypeStruct((N, DE), f32),
            jax.ShapeDtypeStruct((N, DE), f32),
        ],
    )(x, We)

    eaT = edge_attr.T
    t = pl.pallas_call(
        functools.partial(_tc_edges_body, D=D, DE=DE),
        grid=(EBLK,),
        in_specs=[
            pl.BlockSpec((DE, BE), lambda i: (0, i)),
            pl.BlockSpec(We.shape, lambda i: (0, 0)),
            pl.BlockSpec((1, DU), lambda i: (0, 0)),
            pl.BlockSpec((1, DE), lambda i: (0, 0)),
        ],
        out_specs=pl.BlockSpec((BE, DE), lambda i: (i, 0)),
        out_shape=jax.ShapeDtypeStruct((E, DE), f32),
    )(eaT, We, u2, be2)

    info = plsc.get_sparse_core_info()
    NC, NS = info.num_cores, info.num_subcores
    sc_edge = _make_sc_edge(N, E, DE, NC, NS, C=80)
    e_new, p_in, p_out = sc_edge(xs, xd, t, src, dst)

    x_new, u_new = pl.pallas_call(
        functools.partial(_tc_node_body, D=D, DE=DE, DU=DU, N=N, E=E,
                          NBLK=NBLK),
        grid=(NBLK,),
        in_specs=[
            pl.BlockSpec((BN, D), lambda i: (i, 0)),
            pl.BlockSpec((NC, BN, DE), lambda i: (0, i, 0)),
            pl.BlockSpec((NC, BN, DE), lambda i: (0, i, 0)),
            pl.BlockSpec(Wv.shape, lambda i: (0, 0)),
            pl.BlockSpec(Wg.shape, lambda i: (0, 0)),
            pl.BlockSpec((1, DU), lambda i: (0, 0)),
            pl.BlockSpec((1, D), lambda i: (0, 0)),
            pl.BlockSpec((1, DU), lambda i: (0, 0)),
        ],
        out_specs=[
            pl.BlockSpec((BN, D), lambda i: (i, 0)),
            pl.BlockSpec((1, DU), lambda i: (0, 0)),
        ],
        out_shape=[
            jax.ShapeDtypeStruct((N, D), f32),
            jax.ShapeDtypeStruct((1, DU), f32),
        ],
        scratch_shapes=[
            pltpu.VMEM((1, D), f32),
            pltpu.VMEM((1, DE), f32),
        ],
    )(x, p_in, p_out, Wv, Wg, u2, bv2, bg2)

    eT = pl.pallas_call(
        functools.partial(_tc_transpose_body, DE=DE),
        grid=(EBLK,),
        in_specs=[pl.BlockSpec((BE, DE), lambda i: (i, 0))],
        out_specs=pl.BlockSpec((DE, BE), lambda i: (0, i)),
        out_shape=jax.ShapeDtypeStruct((DE, E), f32),
    )(e_new)

    return x_new, eT.T, u_new.reshape(DU)


# speedup vs baseline: 5.7688x; 1.3064x over previous
import functools

import jax
import jax.numpy as jnp
from jax import lax
from jax.experimental import pallas as pl
from jax.experimental.pallas import tpu as pltpu
from jax.experimental.pallas import tpu_sc as plsc


def _tc_nodes_body(x_ref, We_ref, xs_ref, xd_ref, *, D, DE):
    f32 = jnp.float32
    xblk = x_ref[...]
    xs_ref[...] = jnp.dot(xblk, We_ref[DE:DE + D, :], preferred_element_type=f32)
    xd_ref[...] = jnp.dot(xblk, We_ref[DE + D:DE + 2 * D, :],
                          preferred_element_type=f32)


def _tc_transpose_body(e_ref, eT_ref, *, DE):
    f32 = jnp.float32
    ident = (jax.lax.broadcasted_iota(jnp.int32, (DE, DE), 0)
             == jax.lax.broadcasted_iota(jnp.int32, (DE, DE), 1)).astype(f32)
    eT_ref[...] = lax.dot_general(ident, e_ref[...],
                                  (((1,), (1,)), ((), ())),
                                  preferred_element_type=f32)


def _tc_edges_body(eaT_ref, We_ref, u_ref, be_ref, t_ref, *, D, DE):
    f32 = jnp.float32
    ce = jnp.dot(u_ref[...], We_ref[DE + 2 * D:, :],
                 preferred_element_type=f32) + be_ref[...]
    t_ref[...] = lax.dot_general(
        eaT_ref[...], We_ref[:DE, :],
        (((0,), (0,)), ((), ())), preferred_element_type=f32) + ce


def _tc_node_body(x_ref, pin_ref, pout_ref, Wv_ref, Wg_ref, u_ref, bv_ref,
                  bg_ref, xn_ref, un_ref, accx_ref, acce_ref,
                  *, D, DE, DU, N, E, NBLK):
    f32 = jnp.float32
    i = pl.program_id(0)

    @pl.when(i == 0)
    def _():
        accx_ref[...] = jnp.zeros_like(accx_ref)
        acce_ref[...] = jnp.zeros_like(acce_ref)

    aggin = pin_ref[0] + pin_ref[1]
    aggout = pout_ref[0] + pout_ref[1]
    cv = jnp.dot(u_ref[...], Wv_ref[D + 2 * DE:, :],
                 preferred_element_type=f32) + bv_ref[...]
    xn = (jnp.dot(x_ref[...], Wv_ref[:D, :], preferred_element_type=f32)
          + jnp.dot(aggin, Wv_ref[D:D + DE, :], preferred_element_type=f32)
          + jnp.dot(aggout, Wv_ref[D + DE:D + 2 * DE, :],
                    preferred_element_type=f32)
          + cv)
    xn = jnp.maximum(xn, 0.0)
    xn_ref[...] = xn
    accx_ref[...] += jnp.sum(xn, axis=0, keepdims=True)
    acce_ref[...] += jnp.sum(aggin, axis=0, keepdims=True)

    @pl.when(i == NBLK - 1)
    def _():
        gx = jnp.dot(accx_ref[...] * (1.0 / N), Wg_ref[:D, :],
                     preferred_element_type=f32)
        ge = jnp.dot(acce_ref[...] * (1.0 / E), Wg_ref[D:D + DE, :],
                     preferred_element_type=f32)
        gu = jnp.dot(u_ref[...], Wg_ref[D + DE:, :],
                     preferred_element_type=f32)
        un_ref[...] = jnp.maximum(gx + ge + gu + bg_ref[...], 0.0)


def _make_sc_edge(N, E, DE, NC, NS, C):
    NW = NC * NS
    EPW = E // NW
    NCH = EPW // C
    NPT = (N // NS) // 8 * 8
    NPT_LAST = N - NPT * (NS - 1)
    mesh = plsc.VectorSubcoreMesh(core_axis_name="c", subcore_axis_name="s")
    f32 = jnp.float32

    @functools.partial(
        pl.kernel,
        out_type=[
            jax.ShapeDtypeStruct((E, DE), f32),
            jax.ShapeDtypeStruct((NC, N, DE), f32),
            jax.ShapeDtypeStruct((NC, N, DE), f32),
        ],
        mesh=mesh,
        scratch_types=[
            pltpu.VMEM((EPW,), jnp.int32),
            pltpu.VMEM((EPW,), jnp.int32),
            pltpu.VMEM((C, DE), f32),
            pltpu.VMEM((C, DE), f32),
            pltpu.VMEM((C, DE), f32),
            pltpu.VMEM((C, DE), f32),
            pltpu.VMEM((C, DE), f32),
            pltpu.VMEM((C, DE), f32),
            pltpu.VMEM((C, DE), f32),
            pltpu.VMEM((C, DE), f32),
            pltpu.VMEM((NPT_LAST, DE), f32),
            pltpu.VMEM_SHARED((N, DE), f32),
            pltpu.VMEM_SHARED((N, DE), f32),
            pltpu.SemaphoreType.DMA,
            pltpu.SemaphoreType.DMA,
            pltpu.SemaphoreType.DMA,
            pltpu.SemaphoreType.DMA,
            pltpu.SemaphoreType.DMA,
        ],
        compiler_params=pltpu.CompilerParams(use_tc_tiling_on_sc=False),
    )
    def sc_edge(xs_hbm, xd_hbm, t_hbm, src_hbm, dst_hbm,
                e_hbm, pin_hbm, pout_hbm,
                src_all, dst_all, xsb0, xsb1, xdb0, xdb1, tb0, tb1,
                eb0, eb1, zb, agg_in, agg_out,
                semi, semg0, semg1, semw0, semw1):
        cid = lax.axis_index("c")
        sid = lax.axis_index("s")
        wid = sid * NC + cid
        ebase = wid * EPW

        xsb = (xsb0, xsb1)
        xdb = (xdb0, xdb1)
        tb = (tb0, tb1)
        eb = (eb0, eb1)
        semg = (semg0, semg1)
        semw = (semw0, semw1)

        pltpu.async_copy(src_hbm.at[pl.ds(ebase, EPW)], src_all, semi)
        pltpu.async_copy(dst_hbm.at[pl.ds(ebase, EPW)], dst_all, semi)

        zrow = jnp.zeros((16,), f32)

        def zfill(i, _):
            zb[i] = zrow
            return 0

        lax.fori_loop(0, NPT_LAST, zfill, 0, unroll=8)
        nbase = sid * NPT

        @pl.when(sid < NS - 1)
        def _():
            pltpu.sync_copy(zb.at[pl.ds(0, NPT)], agg_in.at[pl.ds(nbase, NPT)])
            pltpu.sync_copy(zb.at[pl.ds(0, NPT)], agg_out.at[pl.ds(nbase, NPT)])

        @pl.when(sid == NS - 1)
        def _():
            pltpu.sync_copy(zb, agg_in.at[pl.ds(nbase, NPT_LAST)])
            pltpu.sync_copy(zb, agg_out.at[pl.ds(nbase, NPT_LAST)])

        pltpu.make_async_copy(src_hbm.at[pl.ds(ebase, EPW)], src_all,
                              semi).wait()
        pltpu.make_async_copy(dst_hbm.at[pl.ds(ebase, EPW)], dst_all,
                              semi).wait()
        plsc.subcore_barrier()

        def issue_gathers(j, s):
            off = pl.multiple_of(j * C, 8)
            pltpu.async_copy(xs_hbm.at[src_all.at[pl.ds(off, C)]],
                             xsb[s], semg[s])
            pltpu.async_copy(xd_hbm.at[dst_all.at[pl.ds(off, C)]],
                             xdb[s], semg[s])
            pltpu.async_copy(t_hbm.at[pl.ds(ebase + off, C)], tb[s], semg[s])

        def drain_gathers(s):
            pltpu.make_async_copy(t_hbm.at[pl.ds(ebase, C)], xsb[s],
                                  semg[s]).wait()
            pltpu.make_async_copy(t_hbm.at[pl.ds(ebase, C)], xdb[s],
                                  semg[s]).wait()
            pltpu.make_async_copy(t_hbm.at[pl.ds(ebase, C)], tb[s],
                                  semg[s]).wait()

        def issue_writes(j, s):
            off = pl.multiple_of(j * C, 8)
            pltpu.sync_copy(eb[s], e_hbm.at[pl.ds(ebase + off, C)])
            pltpu.sync_copy(eb[s], agg_in.at[dst_all.at[pl.ds(off, C)]],
                            add=True)
            pltpu.sync_copy(eb[s], agg_out.at[src_all.at[pl.ds(off, C)]],
                            add=True)

        def drain_writes(s):
            pass

        def compute(s):
            def row(i, _):
                eb[s][i] = jnp.maximum(tb[s][i] + xsb[s][i] + xdb[s][i], 0.0)
                return 0

            lax.fori_loop(0, C, row, 0, unroll=8)

        def stage(j, s):
            @pl.when(j + 1 < NCH)
            def _():
                issue_gathers(j + 1, (s + 1) % 2)

            @pl.when(j < NCH)
            def _():
                drain_gathers(s)
                compute(s)
                issue_writes(j, s)

        issue_gathers(0, 0)

        def body2(jj, _):
            stage(2 * jj, 0)
            stage(2 * jj + 1, 1)
            return 0

        lax.fori_loop(0, (NCH + 1) // 2, body2, 0)

        plsc.subcore_barrier()

        @pl.when(sid < NS - 1)
        def _():
            pltpu.sync_copy(agg_in.at[pl.ds(nbase, NPT)],
                            pin_hbm.at[cid, pl.ds(nbase, NPT)])
            pltpu.sync_copy(agg_out.at[pl.ds(nbase, NPT)],
                            pout_hbm.at[cid, pl.ds(nbase, NPT)])

        @pl.when(sid == NS - 1)
        def _():
            pltpu.sync_copy(agg_in.at[pl.ds(nbase, NPT_LAST)],
                            pin_hbm.at[cid, pl.ds(nbase, NPT_LAST)])
            pltpu.sync_copy(agg_out.at[pl.ds(nbase, NPT_LAST)],
                            pout_hbm.at[cid, pl.ds(nbase, NPT_LAST)])

    return sc_edge


def kernel(x, edge_attr, edge_index, u, We, be, Wv, bv, Wg, bg):
    N, D = x.shape
    E, DE = edge_attr.shape
    DU = u.shape[0]
    f32 = jnp.float32

    src = edge_index[0]
    dst = edge_index[1]
    u2 = u.reshape(1, DU)
    be2 = be.reshape(1, DE)
    bv2 = bv.reshape(1, D)
    bg2 = bg.reshape(1, DU)

    NBLK = 10
    BN = N // NBLK
    BE = 2560
    EBLK = E // BE

    xs, xd = pl.pallas_call(
        functools.partial(_tc_nodes_body, D=D, DE=DE),
        grid=(NBLK,),
        in_specs=[
            pl.BlockSpec((BN, D), lambda i: (i, 0)),
            pl.BlockSpec(We.shape, lambda i: (0, 0)),
        ],
        out_specs=[
            pl.BlockSpec((BN, DE), lambda i: (i, 0)),
            pl.BlockSpec((BN, DE), lambda i: (i, 0)),
        ],
        out_shape=[
            jax.ShapeDtypeStruct((N, DE), f32),
            jax.ShapeDtypeStruct((N, DE), f32),
        ],
    )(x, We)

    eaT = edge_attr.T
    t = pl.pallas_call(
        functools.partial(_tc_edges_body, D=D, DE=DE),
        grid=(EBLK,),
        in_specs=[
            pl.BlockSpec((DE, BE), lambda i: (0, i)),
            pl.BlockSpec(We.shape, lambda i: (0, 0)),
            pl.BlockSpec((1, DU), lambda i: (0, 0)),
            pl.BlockSpec((1, DE), lambda i: (0, 0)),
        ],
        out_specs=pl.BlockSpec((BE, DE), lambda i: (i, 0)),
        out_shape=jax.ShapeDtypeStruct((E, DE), f32),
    )(eaT, We, u2, be2)

    info = plsc.get_sparse_core_info()
    NC, NS = info.num_cores, info.num_subcores
    sc_edge = _make_sc_edge(N, E, DE, NC, NS, C=80)
    e_new, p_in, p_out = sc_edge(xs, xd, t, src, dst)

    x_new, u_new = pl.pallas_call(
        functools.partial(_tc_node_body, D=D, DE=DE, DU=DU, N=N, E=E,
                          NBLK=NBLK),
        grid=(NBLK,),
        in_specs=[
            pl.BlockSpec((BN, D), lambda i: (i, 0)),
            pl.BlockSpec((NC, BN, DE), lambda i: (0, i, 0)),
            pl.BlockSpec((NC, BN, DE), lambda i: (0, i, 0)),
            pl.BlockSpec(Wv.shape, lambda i: (0, 0)),
            pl.BlockSpec(Wg.shape, lambda i: (0, 0)),
            pl.BlockSpec((1, DU), lambda i: (0, 0)),
            pl.BlockSpec((1, D), lambda i: (0, 0)),
            pl.BlockSpec((1, DU), lambda i: (0, 0)),
        ],
        out_specs=[
            pl.BlockSpec((BN, D), lambda i: (i, 0)),
            pl.BlockSpec((1, DU), lambda i: (0, 0)),
        ],
        out_shape=[
            jax.ShapeDtypeStruct((N, D), f32),
            jax.ShapeDtypeStruct((1, DU), f32),
        ],
        scratch_shapes=[
            pltpu.VMEM((1, D), f32),
            pltpu.VMEM((1, DE), f32),
        ],
    )(x, p_in, p_out, Wv, Wg, u2, bv2, bg2)

    eT = pl.pallas_call(
        functools.partial(_tc_transpose_body, DE=DE),
        grid=(EBLK,),
        in_specs=[pl.BlockSpec((BE, DE), lambda i: (i, 0))],
        out_specs=pl.BlockSpec((DE, BE), lambda i: (0, i)),
        out_shape=jax.ShapeDtypeStruct((DE, E), f32),
    )(e_new)

    return x_new, eT.T, u_new.reshape(DU)


# speedup vs baseline: 5.8236x; 1.0095x over previous
import functools

import jax
import jax.numpy as jnp
from jax import lax
from jax.experimental import pallas as pl
from jax.experimental.pallas import tpu as pltpu
from jax.experimental.pallas import tpu_sc as plsc


def _tc_nodes_body(x_ref, We_ref, xs_ref, xd_ref, *, D, DE):
    f32 = jnp.float32
    xblk = x_ref[...]
    xs_ref[...] = jnp.dot(xblk, We_ref[DE:DE + D, :], preferred_element_type=f32)
    xd_ref[...] = jnp.dot(xblk, We_ref[DE + D:DE + 2 * D, :],
                          preferred_element_type=f32)


def _tc_transpose_body(e_ref, eT_ref, *, DE):
    f32 = jnp.float32
    ident = (jax.lax.broadcasted_iota(jnp.int32, (DE, DE), 0)
             == jax.lax.broadcasted_iota(jnp.int32, (DE, DE), 1)).astype(f32)
    eT_ref[...] = lax.dot_general(ident, e_ref[...],
                                  (((1,), (1,)), ((), ())),
                                  preferred_element_type=f32)


def _tc_edges_body(eaT_ref, We_ref, u_ref, be_ref, t_ref, *, D, DE):
    f32 = jnp.float32
    ce = jnp.dot(u_ref[...], We_ref[DE + 2 * D:, :],
                 preferred_element_type=f32) + be_ref[...]
    t_ref[...] = lax.dot_general(
        eaT_ref[...], We_ref[:DE, :],
        (((0,), (0,)), ((), ())), preferred_element_type=f32) + ce


def _tc_node_body(x_ref, pin_ref, pout_ref, Wv_ref, Wg_ref, u_ref, bv_ref,
                  bg_ref, xn_ref, un_ref, accx_ref, acce_ref,
                  *, D, DE, DU, N, E, NBLK):
    f32 = jnp.float32
    i = pl.program_id(0)

    @pl.when(i == 0)
    def _():
        accx_ref[...] = jnp.zeros_like(accx_ref)
        acce_ref[...] = jnp.zeros_like(acce_ref)

    aggin = pin_ref[0] + pin_ref[1]
    aggout = pout_ref[0] + pout_ref[1]
    cv = jnp.dot(u_ref[...], Wv_ref[D + 2 * DE:, :],
                 preferred_element_type=f32) + bv_ref[...]
    xn = (jnp.dot(x_ref[...], Wv_ref[:D, :], preferred_element_type=f32)
          + jnp.dot(aggin, Wv_ref[D:D + DE, :], preferred_element_type=f32)
          + jnp.dot(aggout, Wv_ref[D + DE:D + 2 * DE, :],
                    preferred_element_type=f32)
          + cv)
    xn = jnp.maximum(xn, 0.0)
    xn_ref[...] = xn
    accx_ref[...] += jnp.sum(xn, axis=0, keepdims=True)
    acce_ref[...] += jnp.sum(aggin, axis=0, keepdims=True)

    @pl.when(i == NBLK - 1)
    def _():
        gx = jnp.dot(accx_ref[...] * (1.0 / N), Wg_ref[:D, :],
                     preferred_element_type=f32)
        ge = jnp.dot(acce_ref[...] * (1.0 / E), Wg_ref[D:D + DE, :],
                     preferred_element_type=f32)
        gu = jnp.dot(u_ref[...], Wg_ref[D + DE:, :],
                     preferred_element_type=f32)
        un_ref[...] = jnp.maximum(gx + ge + gu + bg_ref[...], 0.0)


def _make_sc_edge(N, E, DE, NC, NS, C):
    NW = NC * NS
    EPW = E // NW
    NCH = EPW // C
    NPT = (N // NS) // 8 * 8
    NPT_LAST = N - NPT * (NS - 1)
    mesh = plsc.VectorSubcoreMesh(core_axis_name="c", subcore_axis_name="s")
    f32 = jnp.float32

    @functools.partial(
        pl.kernel,
        out_type=[
            jax.ShapeDtypeStruct((E, DE), f32),
            jax.ShapeDtypeStruct((NC, N, DE), f32),
            jax.ShapeDtypeStruct((NC, N, DE), f32),
        ],
        mesh=mesh,
        scratch_types=[
            pltpu.VMEM((EPW,), jnp.int32),
            pltpu.VMEM((EPW,), jnp.int32),
            pltpu.VMEM((C, DE), f32),
            pltpu.VMEM((C, DE), f32),
            pltpu.VMEM((C, DE), f32),
            pltpu.VMEM((C, DE), f32),
            pltpu.VMEM((C, DE), f32),
            pltpu.VMEM((C, DE), f32),
            pltpu.VMEM((C, DE), f32),
            pltpu.VMEM((C, DE), f32),
            pltpu.VMEM((NPT_LAST, DE), f32),
            pltpu.VMEM_SHARED((N, DE), f32),
            pltpu.VMEM_SHARED((N, DE), f32),
            pltpu.SemaphoreType.DMA,
            pltpu.SemaphoreType.DMA,
            pltpu.SemaphoreType.DMA,
            pltpu.SemaphoreType.DMA,
            pltpu.SemaphoreType.DMA,
        ],
        compiler_params=pltpu.CompilerParams(use_tc_tiling_on_sc=False),
    )
    def sc_edge(xs_hbm, xd_hbm, t_hbm, src_hbm, dst_hbm,
                e_hbm, pin_hbm, pout_hbm,
                src_all, dst_all, xsb0, xsb1, xdb0, xdb1, tb0, tb1,
                eb0, eb1, zb, agg_in, agg_out,
                semi, semg0, semg1, semw0, semw1):
        cid = lax.axis_index("c")
        sid = lax.axis_index("s")
        wid = sid * NC + cid
        ebase = wid * EPW

        xsb = (xsb0, xsb1)
        xdb = (xdb0, xdb1)
        tb = (tb0, tb1)
        eb = (eb0, eb1)
        semg = (semg0, semg1)
        semw = (semw0, semw1)

        pltpu.async_copy(src_hbm.at[pl.ds(ebase, EPW)], src_all, semi)
        pltpu.async_copy(dst_hbm.at[pl.ds(ebase, EPW)], dst_all, semi)

        zrow = jnp.zeros((16,), f32)

        def zfill(i, _):
            zb[i] = zrow
            return 0

        lax.fori_loop(0, NPT_LAST, zfill, 0, unroll=8)
        nbase = sid * NPT

        @pl.when(sid < NS - 1)
        def _():
            pltpu.sync_copy(zb.at[pl.ds(0, NPT)], agg_in.at[pl.ds(nbase, NPT)])
            pltpu.sync_copy(zb.at[pl.ds(0, NPT)], agg_out.at[pl.ds(nbase, NPT)])

        @pl.when(sid == NS - 1)
        def _():
            pltpu.sync_copy(zb, agg_in.at[pl.ds(nbase, NPT_LAST)])
            pltpu.sync_copy(zb, agg_out.at[pl.ds(nbase, NPT_LAST)])

        pltpu.make_async_copy(src_hbm.at[pl.ds(ebase, EPW)], src_all,
                              semi).wait()
        pltpu.make_async_copy(dst_hbm.at[pl.ds(ebase, EPW)], dst_all,
                              semi).wait()
        plsc.subcore_barrier()

        def issue_gathers(j, s):
            off = pl.multiple_of(j * C, 8)
            pltpu.async_copy(xs_hbm.at[src_all.at[pl.ds(off, C)]],
                             xsb[s], semg[s])
            pltpu.async_copy(xd_hbm.at[dst_all.at[pl.ds(off, C)]],
                             xdb[s], semg[s])
            pltpu.async_copy(t_hbm.at[pl.ds(ebase + off, C)], tb[s], semg[s])

        def drain_gathers(s):
            pltpu.make_async_copy(t_hbm.at[pl.ds(ebase, C)], xsb[s],
                                  semg[s]).wait()
            pltpu.make_async_copy(t_hbm.at[pl.ds(ebase, C)], xdb[s],
                                  semg[s]).wait()
            pltpu.make_async_copy(t_hbm.at[pl.ds(ebase, C)], tb[s],
                                  semg[s]).wait()

        def issue_writes(j, s):
            off = pl.multiple_of(j * C, 8)
            pltpu.async_copy(eb[s], e_hbm.at[pl.ds(ebase + off, C)], semw[s])
            pltpu.sync_copy(eb[s], agg_in.at[dst_all.at[pl.ds(off, C)]],
                            add=True)
            pltpu.sync_copy(eb[s], agg_out.at[src_all.at[pl.ds(off, C)]],
                            add=True)

        def drain_writes(s):
            pltpu.make_async_copy(eb[s], e_hbm.at[pl.ds(ebase, C)],
                                  semw[s]).wait()

        def compute(s):
            def row(i, _):
                eb[s][i] = jnp.maximum(tb[s][i] + xsb[s][i] + xdb[s][i], 0.0)
                return 0

            lax.fori_loop(0, C, row, 0, unroll=8)

        def stage(j, s):
            @pl.when(jnp.logical_and(j >= 2, j < NCH + 2))
            def _():
                drain_writes(s)

            @pl.when(j + 1 < NCH)
            def _():
                issue_gathers(j + 1, (s + 1) % 2)

            @pl.when(j < NCH)
            def _():
                drain_gathers(s)
                compute(s)
                issue_writes(j, s)

        issue_gathers(0, 0)

        def body2(jj, _):
            stage(2 * jj, 0)
            stage(2 * jj + 1, 1)
            return 0

        lax.fori_loop(0, (NCH + 3) // 2, body2, 0)

        plsc.subcore_barrier()

        @pl.when(sid < NS - 1)
        def _():
            pltpu.sync_copy(agg_in.at[pl.ds(nbase, NPT)],
                            pin_hbm.at[cid, pl.ds(nbase, NPT)])
            pltpu.sync_copy(agg_out.at[pl.ds(nbase, NPT)],
                            pout_hbm.at[cid, pl.ds(nbase, NPT)])

        @pl.when(sid == NS - 1)
        def _():
            pltpu.sync_copy(agg_in.at[pl.ds(nbase, NPT_LAST)],
                            pin_hbm.at[cid, pl.ds(nbase, NPT_LAST)])
            pltpu.sync_copy(agg_out.at[pl.ds(nbase, NPT_LAST)],
                            pout_hbm.at[cid, pl.ds(nbase, NPT_LAST)])

    return sc_edge


def kernel(x, edge_attr, edge_index, u, We, be, Wv, bv, Wg, bg):
    N, D = x.shape
    E, DE = edge_attr.shape
    DU = u.shape[0]
    f32 = jnp.float32

    src = edge_index[0]
    dst = edge_index[1]
    u2 = u.reshape(1, DU)
    be2 = be.reshape(1, DE)
    bv2 = bv.reshape(1, D)
    bg2 = bg.reshape(1, DU)

    NBLK = 10
    BN = N // NBLK
    BE = 2560
    EBLK = E // BE

    xs, xd = pl.pallas_call(
        functools.partial(_tc_nodes_body, D=D, DE=DE),
        grid=(NBLK,),
        in_specs=[
            pl.BlockSpec((BN, D), lambda i: (i, 0)),
            pl.BlockSpec(We.shape, lambda i: (0, 0)),
        ],
        out_specs=[
            pl.BlockSpec((BN, DE), lambda i: (i, 0)),
            pl.BlockSpec((BN, DE), lambda i: (i, 0)),
        ],
        out_shape=[
            jax.ShapeDtypeStruct((N, DE), f32),
            jax.ShapeDtypeStruct((N, DE), f32),
        ],
    )(x, We)

    eaT = edge_attr.T
    t = pl.pallas_call(
        functools.partial(_tc_edges_body, D=D, DE=DE),
        grid=(EBLK,),
        in_specs=[
            pl.BlockSpec((DE, BE), lambda i: (0, i)),
            pl.BlockSpec(We.shape, lambda i: (0, 0)),
            pl.BlockSpec((1, DU), lambda i: (0, 0)),
            pl.BlockSpec((1, DE), lambda i: (0, 0)),
        ],
        out_specs=pl.BlockSpec((BE, DE), lambda i: (i, 0)),
        out_shape=jax.ShapeDtypeStruct((E, DE), f32),
    )(eaT, We, u2, be2)

    info = plsc.get_sparse_core_info()
    NC, NS = info.num_cores, info.num_subcores
    sc_edge = _make_sc_edge(N, E, DE, NC, NS, C=80)
    e_new, p_in, p_out = sc_edge(xs, xd, t, src, dst)

    x_new, u_new = pl.pallas_call(
        functools.partial(_tc_node_body, D=D, DE=DE, DU=DU, N=N, E=E,
                          NBLK=NBLK),
        grid=(NBLK,),
        in_specs=[
            pl.BlockSpec((BN, D), lambda i: (i, 0)),
            pl.BlockSpec((NC, BN, DE), lambda i: (0, i, 0)),
            pl.BlockSpec((NC, BN, DE), lambda i: (0, i, 0)),
            pl.BlockSpec(Wv.shape, lambda i: (0, 0)),
            pl.BlockSpec(Wg.shape, lambda i: (0, 0)),
            pl.BlockSpec((1, DU), lambda i: (0, 0)),
            pl.BlockSpec((1, D), lambda i: (0, 0)),
            pl.BlockSpec((1, DU), lambda i: (0, 0)),
        ],
        out_specs=[
            pl.BlockSpec((BN, D), lambda i: (i, 0)),
            pl.BlockSpec((1, DU), lambda i: (0, 0)),
        ],
        out_shape=[
            jax.ShapeDtypeStruct((N, D), f32),
            jax.ShapeDtypeStruct((1, DU), f32),
        ],
        scratch_shapes=[
            pltpu.VMEM((1, D), f32),
            pltpu.VMEM((1, DE), f32),
        ],
    )(x, p_in, p_out, Wv, Wg, u2, bv2, bg2)

    eT = pl.pallas_call(
        functools.partial(_tc_transpose_body, DE=DE),
        grid=(EBLK,),
        in_specs=[pl.BlockSpec((BE, DE), lambda i: (i, 0))],
        out_specs=pl.BlockSpec((DE, BE), lambda i: (0, i)),
        out_shape=jax.ShapeDtypeStruct((DE, E), f32),
    )(e_new)

    return x_new, eT.T, u_new.reshape(DU)


# speedup vs baseline: 9.3091x; 1.5985x over previous
import functools

import jax
import jax.numpy as jnp
from jax import lax
from jax.experimental import pallas as pl
from jax.experimental.pallas import tpu as pltpu
from jax.experimental.pallas import tpu_sc as plsc


def _tc_nodes_body(x_ref, We_ref, xs_ref, xd_ref, *, D, DE):
    f32 = jnp.float32
    xblk = x_ref[...]
    xs_ref[...] = jnp.dot(xblk, We_ref[DE:DE + D, :], preferred_element_type=f32)
    xd_ref[...] = jnp.dot(xblk, We_ref[DE + D:DE + 2 * D, :],
                          preferred_element_type=f32)


def _tc_edges_body(eaT_ref, We_ref, u_ref, be_ref, tT_ref, *, D, DE):
    f32 = jnp.float32
    ident = (jax.lax.broadcasted_iota(jnp.int32, (DE, DE), 0)
             == jax.lax.broadcasted_iota(jnp.int32, (DE, DE), 1)).astype(f32)
    ceT = lax.dot_general(We_ref[DE + 2 * D:, :], u_ref[...],
                          (((0,), (1,)), ((), ())),
                          preferred_element_type=f32)
    beT = lax.dot_general(ident, be_ref[...],
                          (((1,), (1,)), ((), ())),
                          preferred_element_type=f32)
    tT_ref[...] = lax.dot_general(
        We_ref[:DE, :], eaT_ref[...],
        (((0,), (0,)), ((), ())), preferred_element_type=f32) + ceT + beT


def _tc_node_body(x_ref, pin_ref, pout_ref, Wv_ref, Wg_ref, u_ref, bv_ref,
                  bg_ref, xn_ref, un_ref, accx_ref, acce_ref,
                  *, D, DE, DU, N, E, NBLK):
    f32 = jnp.float32
    i = pl.program_id(0)

    @pl.when(i == 0)
    def _():
        accx_ref[...] = jnp.zeros_like(accx_ref)
        acce_ref[...] = jnp.zeros_like(acce_ref)

    aggin = pin_ref[0] + pin_ref[1]
    aggout = pout_ref[0] + pout_ref[1]
    cv = jnp.dot(u_ref[...], Wv_ref[D + 2 * DE:, :],
                 preferred_element_type=f32) + bv_ref[...]
    xn = (jnp.dot(x_ref[...], Wv_ref[:D, :], preferred_element_type=f32)
          + jnp.dot(aggin, Wv_ref[D:D + DE, :], preferred_element_type=f32)
          + jnp.dot(aggout, Wv_ref[D + DE:D + 2 * DE, :],
                    preferred_element_type=f32)
          + cv)
    xn = jnp.maximum(xn, 0.0)
    xn_ref[...] = xn
    accx_ref[...] += jnp.sum(xn, axis=0, keepdims=True)
    acce_ref[...] += jnp.sum(aggin, axis=0, keepdims=True)

    @pl.when(i == NBLK - 1)
    def _():
        gx = jnp.dot(accx_ref[...] * (1.0 / N), Wg_ref[:D, :],
                     preferred_element_type=f32)
        ge = jnp.dot(acce_ref[...] * (1.0 / E), Wg_ref[D:D + DE, :],
                     preferred_element_type=f32)
        gu = jnp.dot(u_ref[...], Wg_ref[D + DE:, :],
                     preferred_element_type=f32)
        un_ref[...] = jnp.maximum(gx + ge + gu + bg_ref[...], 0.0)


def _make_sc_edge(N, E, DE, NC, NS, C, W):
    NW = NC * NS
    EPW = E // NW
    CPW = W // C
    NWIN = EPW // W
    NPT = (N // NS) // 8 * 8
    NPT_LAST = N - NPT * (NS - 1)
    mesh = plsc.VectorSubcoreMesh(core_axis_name="c", subcore_axis_name="s")
    f32 = jnp.float32

    @functools.partial(
        pl.kernel,
        out_type=[
            jax.ShapeDtypeStruct((DE, E), f32),
            jax.ShapeDtypeStruct((NC, N, DE), f32),
            jax.ShapeDtypeStruct((NC, N, DE), f32),
        ],
        mesh=mesh,
        scratch_types=[
            pltpu.VMEM((EPW,), jnp.int32),
            pltpu.VMEM((EPW,), jnp.int32),
            pltpu.VMEM((C, DE), f32),
            pltpu.VMEM((C, DE), f32),
            pltpu.VMEM((C, DE), f32),
            pltpu.VMEM((C, DE), f32),
            pltpu.VMEM((C, DE), f32),
            pltpu.VMEM((C, DE), f32),
            pltpu.VMEM((DE * W,), f32),
            pltpu.VMEM((DE * W,), f32),
            pltpu.VMEM((NPT_LAST, DE), f32),
            pltpu.VMEM_SHARED((N, DE), f32),
            pltpu.VMEM_SHARED((N, DE), f32),
            pltpu.SemaphoreType.DMA,
            pltpu.SemaphoreType.DMA,
            pltpu.SemaphoreType.DMA,
            pltpu.SemaphoreType.DMA,
            pltpu.SemaphoreType.DMA,
        ],
        compiler_params=pltpu.CompilerParams(use_tc_tiling_on_sc=False,
                                             needs_layout_passes=False),
    )
    def sc_edge(xs_hbm, xd_hbm, tT_hbm, src_hbm, dst_hbm,
                eT_hbm, pin_hbm, pout_hbm,
                src_all, dst_all, xsb0, xsb1, xdb0, xdb1,
                eb0, eb1, tbT, ebT, zb, agg_in, agg_out,
                semi, semg0, semg1, semT, semF):
        cid = lax.axis_index("c")
        sid = lax.axis_index("s")
        wid = sid * NC + cid
        ebase = wid * EPW

        xsb = (xsb0, xsb1)
        xdb = (xdb0, xdb1)
        eb = (eb0, eb1)
        semg = (semg0, semg1)
        ivW = lax.iota(jnp.int32, 16) * W

        pltpu.async_copy(src_hbm.at[pl.ds(ebase, EPW)], src_all, semi)
        pltpu.async_copy(dst_hbm.at[pl.ds(ebase, EPW)], dst_all, semi)

        zrow = jnp.zeros((16,), f32)

        def zfill(i, _):
            zb[i] = zrow
            return 0

        lax.fori_loop(0, NPT_LAST, zfill, 0, unroll=8)
        nbase = sid * NPT

        @pl.when(sid < NS - 1)
        def _():
            pltpu.sync_copy(zb.at[pl.ds(0, NPT)], agg_in.at[pl.ds(nbase, NPT)])
            pltpu.sync_copy(zb.at[pl.ds(0, NPT)], agg_out.at[pl.ds(nbase, NPT)])

        @pl.when(sid == NS - 1)
        def _():
            pltpu.sync_copy(zb, agg_in.at[pl.ds(nbase, NPT_LAST)])
            pltpu.sync_copy(zb, agg_out.at[pl.ds(nbase, NPT_LAST)])

        pltpu.make_async_copy(src_hbm.at[pl.ds(ebase, EPW)], src_all,
                              semi).wait()
        pltpu.make_async_copy(dst_hbm.at[pl.ds(ebase, EPW)], dst_all,
                              semi).wait()
        plsc.subcore_barrier()

        def issue_gathers(j, s):
            off = pl.multiple_of(j * C, 8)
            pltpu.async_copy(xs_hbm.at[src_all.at[pl.ds(off, C)]],
                             xsb[s], semg[s])
            pltpu.async_copy(xd_hbm.at[dst_all.at[pl.ds(off, C)]],
                             xdb[s], semg[s])

        def drain_gathers(s):
            pltpu.make_async_copy(xs_hbm.at[pl.ds(0, C)], xsb[s],
                                  semg[s]).wait()
            pltpu.make_async_copy(xd_hbm.at[pl.ds(0, C)], xdb[s],
                                  semg[s]).wait()

        def compute(c, s):
            def row8(q, _):
                for a in range(8):
                    i = q * 8 + a
                    idxv = ivW + (c * C + i)
                    tv = plsc.load_gather(tbT, [idxv])
                    v = jnp.maximum(tv + xsb[s][i] + xdb[s][i], 0.0)
                    eb[s][i] = v
                    plsc.store_scatter(ebT, [idxv], v)
                return 0

            lax.fori_loop(0, C // 8, row8, 0)

        def issue_writes(j, s):
            off = pl.multiple_of(j * C, 8)
            pltpu.sync_copy(eb[s], agg_in.at[dst_all.at[pl.ds(off, C)]],
                            add=True)
            pltpu.sync_copy(eb[s], agg_out.at[src_all.at[pl.ds(off, C)]],
                            add=True)

        for w in range(NWIN):
            wbase = ebase + w * W
            for f in range(DE):
                pltpu.async_copy(tT_hbm.at[f, pl.ds(wbase, W)],
                                 tbT.at[pl.ds(f * W, W)], semT)
            if w > 0:
                pwbase = ebase + (w - 1) * W
                for f in range(DE):
                    pltpu.make_async_copy(ebT.at[pl.ds(f * W, W)],
                                          eT_hbm.at[f, pl.ds(pwbase, W)],
                                          semF).wait()
            for f in range(DE):
                pltpu.make_async_copy(tT_hbm.at[f, pl.ds(wbase, W)],
                                      tbT.at[pl.ds(f * W, W)], semT).wait()

            gbase = w * CPW

            def stage(c, s):
                @pl.when(c + 1 < CPW)
                def _():
                    issue_gathers(gbase + c + 1, (s + 1) % 2)

                @pl.when(c < CPW)
                def _():
                    drain_gathers(s)
                    compute(c, s)
                    issue_writes(gbase + c, s)

            issue_gathers(gbase, 0)

            def body2(cc, _):
                stage(2 * cc, 0)
                stage(2 * cc + 1, 1)
                return 0

            lax.fori_loop(0, (CPW + 1) // 2, body2, 0)

            for f in range(DE):
                pltpu.async_copy(ebT.at[pl.ds(f * W, W)],
                                 eT_hbm.at[f, pl.ds(wbase, W)], semF)

        lwbase = ebase + (NWIN - 1) * W
        for f in range(DE):
            pltpu.make_async_copy(ebT.at[pl.ds(f * W, W)],
                                  eT_hbm.at[f, pl.ds(lwbase, W)], semF).wait()

        plsc.subcore_barrier()

        @pl.when(sid < NS - 1)
        def _():
            pltpu.sync_copy(agg_in.at[pl.ds(nbase, NPT)],
                            pin_hbm.at[cid, pl.ds(nbase, NPT)])
            pltpu.sync_copy(agg_out.at[pl.ds(nbase, NPT)],
                            pout_hbm.at[cid, pl.ds(nbase, NPT)])

        @pl.when(sid == NS - 1)
        def _():
            pltpu.sync_copy(agg_in.at[pl.ds(nbase, NPT_LAST)],
                            pin_hbm.at[cid, pl.ds(nbase, NPT_LAST)])
            pltpu.sync_copy(agg_out.at[pl.ds(nbase, NPT_LAST)],
                            pout_hbm.at[cid, pl.ds(nbase, NPT_LAST)])

    return sc_edge


def kernel(x, edge_attr, edge_index, u, We, be, Wv, bv, Wg, bg):
    N, D = x.shape
    E, DE = edge_attr.shape
    DU = u.shape[0]
    f32 = jnp.float32

    src = edge_index[0]
    dst = edge_index[1]
    u2 = u.reshape(1, DU)
    be2 = be.reshape(1, DE)
    bv2 = bv.reshape(1, D)
    bg2 = bg.reshape(1, DU)

    NBLK = 10
    BN = N // NBLK
    BE = 2560
    EBLK = E // BE

    xs, xd = pl.pallas_call(
        functools.partial(_tc_nodes_body, D=D, DE=DE),
        grid=(NBLK,),
        in_specs=[
            pl.BlockSpec((BN, D), lambda i: (i, 0)),
            pl.BlockSpec(We.shape, lambda i: (0, 0)),
        ],
        out_specs=[
            pl.BlockSpec((BN, DE), lambda i: (i, 0)),
            pl.BlockSpec((BN, DE), lambda i: (i, 0)),
        ],
        out_shape=[
            jax.ShapeDtypeStruct((N, DE), f32),
            jax.ShapeDtypeStruct((N, DE), f32),
        ],
    )(x, We)

    eaT = edge_attr.T
    tT = pl.pallas_call(
        functools.partial(_tc_edges_body, D=D, DE=DE),
        grid=(EBLK,),
        in_specs=[
            pl.BlockSpec((DE, BE), lambda i: (0, i)),
            pl.BlockSpec(We.shape, lambda i: (0, 0)),
            pl.BlockSpec((1, DU), lambda i: (0, 0)),
            pl.BlockSpec((1, DE), lambda i: (0, 0)),
        ],
        out_specs=pl.BlockSpec((DE, BE), lambda i: (0, i)),
        out_shape=jax.ShapeDtypeStruct((DE, E), f32),
    )(eaT, We, u2, be2)

    info = plsc.get_sparse_core_info()
    NC, NS = info.num_cores, info.num_subcores
    sc_edge = _make_sc_edge(N, E, DE, NC, NS, C=80, W=2000)
    eT, p_in, p_out = sc_edge(xs, xd, tT, src, dst)

    x_new, u_new = pl.pallas_call(
        functools.partial(_tc_node_body, D=D, DE=DE, DU=DU, N=N, E=E,
                          NBLK=NBLK),
        grid=(NBLK,),
        in_specs=[
            pl.BlockSpec((BN, D), lambda i: (i, 0)),
            pl.BlockSpec((NC, BN, DE), lambda i: (0, i, 0)),
            pl.BlockSpec((NC, BN, DE), lambda i: (0, i, 0)),
            pl.BlockSpec(Wv.shape, lambda i: (0, 0)),
            pl.BlockSpec(Wg.shape, lambda i: (0, 0)),
            pl.BlockSpec((1, DU), lambda i: (0, 0)),
            pl.BlockSpec((1, D), lambda i: (0, 0)),
            pl.BlockSpec((1, DU), lambda i: (0, 0)),
        ],
        out_specs=[
            pl.BlockSpec((BN, D), lambda i: (i, 0)),
            pl.BlockSpec((1, DU), lambda i: (0, 0)),
        ],
        out_shape=[
            jax.ShapeDtypeStruct((N, D), f32),
            jax.ShapeDtypeStruct((1, DU), f32),
        ],
        scratch_shapes=[
            pltpu.VMEM((1, D), f32),
            pltpu.VMEM((1, DE), f32),
        ],
    )(x, p_in, p_out, Wv, Wg, u2, bv2, bg2)

    return x_new, eT.T, u_new.reshape(DU)


# speedup vs baseline: 13.2172x; 1.4198x over previous
import functools

import jax
import jax.numpy as jnp
from jax import lax
from jax.experimental import pallas as pl
from jax.experimental.pallas import tpu as pltpu
from jax.experimental.pallas import tpu_sc as plsc


def _tc_nodes_body(x8_ref, W8s_ref, W8d_ref, xs_ref, xd_ref):
    f32 = jnp.float32
    x8 = x8_ref[...]
    xs_ref[...] = jnp.dot(x8, W8s_ref[...], preferred_element_type=f32)
    xd_ref[...] = jnp.dot(x8, W8d_ref[...], preferred_element_type=f32)


def _tc_edges_body(eaT_ref, We_ref, u_ref, be_ref, tT_ref, *, D, DE):
    f32 = jnp.float32
    ident = (jax.lax.broadcasted_iota(jnp.int32, (DE, DE), 0)
             == jax.lax.broadcasted_iota(jnp.int32, (DE, DE), 1)).astype(f32)
    ceT = lax.dot_general(We_ref[DE + 2 * D:, :], u_ref[...],
                          (((0,), (1,)), ((), ())),
                          preferred_element_type=f32)
    beT = lax.dot_general(ident, be_ref[...],
                          (((1,), (1,)), ((), ())),
                          preferred_element_type=f32)
    tT_ref[...] = lax.dot_general(
        We_ref[:DE, :], eaT_ref[...],
        (((0,), (0,)), ((), ())), preferred_element_type=f32) + ceT + beT


def _tc_node_body(x_ref, pin_ref, pout_ref, Wv_ref, Wg_ref, u_ref, bv_ref,
                  bg_ref, xn_ref, un_ref, accx_ref, acce_ref,
                  *, D, DE, DU, N, E, NBLK):
    f32 = jnp.float32
    i = pl.program_id(0)

    @pl.when(i == 0)
    def _():
        accx_ref[...] = jnp.zeros_like(accx_ref)
        acce_ref[...] = jnp.zeros_like(acce_ref)

    aggin = pin_ref[0] + pin_ref[1]
    aggout = pout_ref[0] + pout_ref[1]
    cv = jnp.dot(u_ref[...], Wv_ref[D + 2 * DE:, :],
                 preferred_element_type=f32) + bv_ref[...]
    xn = (jnp.dot(x_ref[...], Wv_ref[:D, :], preferred_element_type=f32)
          + jnp.dot(aggin, Wv_ref[D:D + DE, :], preferred_element_type=f32)
          + jnp.dot(aggout, Wv_ref[D + DE:D + 2 * DE, :],
                    preferred_element_type=f32)
          + cv)
    xn = jnp.maximum(xn, 0.0)
    xn_ref[...] = xn
    accx_ref[...] += jnp.sum(xn, axis=0, keepdims=True)
    acce_ref[...] += jnp.sum(aggin, axis=0, keepdims=True)

    @pl.when(i == NBLK - 1)
    def _():
        gx = jnp.dot(accx_ref[...] * (1.0 / N), Wg_ref[:D, :],
                     preferred_element_type=f32)
        ge = jnp.dot(acce_ref[...] * (1.0 / E), Wg_ref[D:D + DE, :],
                     preferred_element_type=f32)
        gu = jnp.dot(u_ref[...], Wg_ref[D + DE:, :],
                     preferred_element_type=f32)
        un_ref[...] = jnp.maximum(gx + ge + gu + bg_ref[...], 0.0)


def _make_sc_edge(N, E, DE, NC, NS, C, W):
    NW = NC * NS
    EPW = E // NW
    CPW = W // C
    NWIN = EPW // W
    NPT = (N // NS) // 8 * 8
    NPT_LAST = N - NPT * (NS - 1)
    mesh = plsc.VectorSubcoreMesh(core_axis_name="c", subcore_axis_name="s")
    f32 = jnp.float32

    @functools.partial(
        pl.kernel,
        out_type=[
            jax.ShapeDtypeStruct((DE, E), f32),
            jax.ShapeDtypeStruct((NC, N, DE), f32),
            jax.ShapeDtypeStruct((NC, N, DE), f32),
        ],
        mesh=mesh,
        scratch_types=[
            pltpu.VMEM((EPW,), jnp.int32),
            pltpu.VMEM((EPW,), jnp.int32),
            pltpu.VMEM((C, DE), f32),
            pltpu.VMEM((C, DE), f32),
            pltpu.VMEM((C, DE), f32),
            pltpu.VMEM((C, DE), f32),
            pltpu.VMEM((C, DE), f32),
            pltpu.VMEM((C, DE), f32),
            pltpu.VMEM((DE * W,), f32),
            pltpu.VMEM((DE * W,), f32),
            pltpu.VMEM((NPT_LAST, DE), f32),
            pltpu.VMEM_SHARED((N, DE), f32),
            pltpu.VMEM_SHARED((N, DE), f32),
            pltpu.SemaphoreType.DMA,
            pltpu.SemaphoreType.DMA,
            pltpu.SemaphoreType.DMA,
            pltpu.SemaphoreType.DMA,
            pltpu.SemaphoreType.DMA,
        ],
        compiler_params=pltpu.CompilerParams(use_tc_tiling_on_sc=False,
                                             needs_layout_passes=False),
    )
    def sc_edge(xs_hbm, xd_hbm, tT_hbm, src_hbm, dst_hbm,
                eT_hbm, pin_hbm, pout_hbm,
                src_all, dst_all, xsb0, xsb1, xdb0, xdb1,
                eb0, eb1, tbT, ebT, zb, agg_in, agg_out,
                semi, semg0, semg1, semT, semF):
        cid = lax.axis_index("c")
        sid = lax.axis_index("s")
        wid = sid * NC + cid
        ebase = wid * EPW

        xsb = (xsb0, xsb1)
        xdb = (xdb0, xdb1)
        eb = (eb0, eb1)
        semg = (semg0, semg1)
        ivW = lax.iota(jnp.int32, 16) * W

        pltpu.async_copy(src_hbm.at[pl.ds(ebase, EPW)], src_all, semi)
        pltpu.async_copy(dst_hbm.at[pl.ds(ebase, EPW)], dst_all, semi)

        zrow = jnp.zeros((16,), f32)

        def zfill(i, _):
            zb[i] = zrow
            return 0

        lax.fori_loop(0, NPT_LAST, zfill, 0, unroll=8)
        nbase = sid * NPT

        @pl.when(sid < NS - 1)
        def _():
            pltpu.sync_copy(zb.at[pl.ds(0, NPT)], agg_in.at[pl.ds(nbase, NPT)])
            pltpu.sync_copy(zb.at[pl.ds(0, NPT)], agg_out.at[pl.ds(nbase, NPT)])

        @pl.when(sid == NS - 1)
        def _():
            pltpu.sync_copy(zb, agg_in.at[pl.ds(nbase, NPT_LAST)])
            pltpu.sync_copy(zb, agg_out.at[pl.ds(nbase, NPT_LAST)])

        pltpu.make_async_copy(src_hbm.at[pl.ds(ebase, EPW)], src_all,
                              semi).wait()
        pltpu.make_async_copy(dst_hbm.at[pl.ds(ebase, EPW)], dst_all,
                              semi).wait()
        plsc.subcore_barrier()

        def issue_gathers(j, s):
            off = pl.multiple_of(j * C, 8)
            pltpu.async_copy(xs_hbm.at[src_all.at[pl.ds(off, C)]],
                             xsb[s], semg[s])
            pltpu.async_copy(xd_hbm.at[dst_all.at[pl.ds(off, C)]],
                             xdb[s], semg[s])

        def drain_gathers(s):
            pltpu.make_async_copy(xs_hbm.at[pl.ds(0, C)], xsb[s],
                                  semg[s]).wait()
            pltpu.make_async_copy(xd_hbm.at[pl.ds(0, C)], xdb[s],
                                  semg[s]).wait()

        def compute(c, s):
            @plsc.parallel_loop(0, C // 8, unroll=2)
            def _(q):
                for a in range(8):
                    i = q * 8 + a
                    idxv = ivW + (c * C + i)
                    tv = plsc.load_gather(tbT, [idxv])
                    v = jnp.maximum(tv + xsb[s][i] + xdb[s][i], 0.0)
                    eb[s][i] = v
                    plsc.store_scatter(ebT, [idxv], v)

        def issue_writes(j, s):
            off = pl.multiple_of(j * C, 8)
            pltpu.sync_copy(eb[s], agg_in.at[dst_all.at[pl.ds(off, C)]],
                            add=True)
            pltpu.sync_copy(eb[s], agg_out.at[src_all.at[pl.ds(off, C)]],
                            add=True)

        for w in range(NWIN):
            wbase = ebase + w * W
            for f in range(DE):
                pltpu.async_copy(tT_hbm.at[f, pl.ds(wbase, W)],
                                 tbT.at[pl.ds(f * W, W)], semT)
            if w > 0:
                pwbase = ebase + (w - 1) * W
                for f in range(DE):
                    pltpu.make_async_copy(ebT.at[pl.ds(f * W, W)],
                                          eT_hbm.at[f, pl.ds(pwbase, W)],
                                          semF).wait()
            for f in range(DE):
                pltpu.make_async_copy(tT_hbm.at[f, pl.ds(wbase, W)],
                                      tbT.at[pl.ds(f * W, W)], semT).wait()

            gbase = w * CPW

            def stage(c, s):
                @pl.when(c + 1 < CPW)
                def _():
                    issue_gathers(gbase + c + 1, (s + 1) % 2)

                @pl.when(c < CPW)
                def _():
                    drain_gathers(s)
                    compute(c, s)
                    issue_writes(gbase + c, s)

            issue_gathers(gbase, 0)

            def body2(cc, _):
                stage(2 * cc, 0)
                stage(2 * cc + 1, 1)
                return 0

            lax.fori_loop(0, (CPW + 1) // 2, body2, 0)

            for f in range(DE):
                pltpu.async_copy(ebT.at[pl.ds(f * W, W)],
                                 eT_hbm.at[f, pl.ds(wbase, W)], semF)

        lwbase = ebase + (NWIN - 1) * W
        for f in range(DE):
            pltpu.make_async_copy(ebT.at[pl.ds(f * W, W)],
                                  eT_hbm.at[f, pl.ds(lwbase, W)], semF).wait()

        plsc.subcore_barrier()

        @pl.when(sid < NS - 1)
        def _():
            pltpu.sync_copy(agg_in.at[pl.ds(nbase, NPT)],
                            pin_hbm.at[cid, pl.ds(nbase, NPT)])
            pltpu.sync_copy(agg_out.at[pl.ds(nbase, NPT)],
                            pout_hbm.at[cid, pl.ds(nbase, NPT)])

        @pl.when(sid == NS - 1)
        def _():
            pltpu.sync_copy(agg_in.at[pl.ds(nbase, NPT_LAST)],
                            pin_hbm.at[cid, pl.ds(nbase, NPT_LAST)])
            pltpu.sync_copy(agg_out.at[pl.ds(nbase, NPT_LAST)],
                            pout_hbm.at[cid, pl.ds(nbase, NPT_LAST)])

    return sc_edge


def kernel(x, edge_attr, edge_index, u, We, be, Wv, bv, Wg, bg):
    N, D = x.shape
    E, DE = edge_attr.shape
    DU = u.shape[0]
    f32 = jnp.float32

    src = edge_index[0]
    dst = edge_index[1]
    u2 = u.reshape(1, DU)
    be2 = be.reshape(1, DE)
    bv2 = bv.reshape(1, D)
    bg2 = bg.reshape(1, DU)

    NBLK = 10
    BN = N // NBLK
    BE = 6400
    EBLK = E // BE

    x8 = x.reshape(N // 8, 8 * D)
    eye8 = jnp.eye(8, dtype=f32)
    W8s = jnp.kron(eye8, We[DE:DE + D, :])
    W8d = jnp.kron(eye8, We[DE + D:DE + 2 * D, :])
    xsp, xdp = pl.pallas_call(
        _tc_nodes_body,
        grid=(1,),
        in_specs=[
            pl.BlockSpec((N // 8, 8 * D), lambda i: (0, 0)),
            pl.BlockSpec((8 * D, 8 * DE), lambda i: (0, 0)),
            pl.BlockSpec((8 * D, 8 * DE), lambda i: (0, 0)),
        ],
        out_specs=[
            pl.BlockSpec((N // 8, 8 * DE), lambda i: (0, 0)),
            pl.BlockSpec((N // 8, 8 * DE), lambda i: (0, 0)),
        ],
        out_shape=[
            jax.ShapeDtypeStruct((N // 8, 8 * DE), f32),
            jax.ShapeDtypeStruct((N // 8, 8 * DE), f32),
        ],
    )(x8, W8s, W8d)
    xs = xsp.reshape(N, DE)
    xd = xdp.reshape(N, DE)

    eaT = edge_attr.T
    tT = pl.pallas_call(
        functools.partial(_tc_edges_body, D=D, DE=DE),
        grid=(EBLK,),
        in_specs=[
            pl.BlockSpec((DE, BE), lambda i: (0, i)),
            pl.BlockSpec(We.shape, lambda i: (0, 0)),
            pl.BlockSpec((1, DU), lambda i: (0, 0)),
            pl.BlockSpec((1, DE), lambda i: (0, 0)),
        ],
        out_specs=pl.BlockSpec((DE, BE), lambda i: (0, i)),
        out_shape=jax.ShapeDtypeStruct((DE, E), f32),
    )(eaT, We, u2, be2)

    info = plsc.get_sparse_core_info()
    NC, NS = info.num_cores, info.num_subcores
    sc_edge = _make_sc_edge(N, E, DE, NC, NS, C=80, W=2000)
    eT, p_in, p_out = sc_edge(xs, xd, tT, src, dst)

    x_new, u_new = pl.pallas_call(
        functools.partial(_tc_node_body, D=D, DE=DE, DU=DU, N=N, E=E,
                          NBLK=NBLK),
        grid=(NBLK,),
        in_specs=[
            pl.BlockSpec((BN, D), lambda i: (i, 0)),
            pl.BlockSpec((NC, BN, DE), lambda i: (0, i, 0)),
            pl.BlockSpec((NC, BN, DE), lambda i: (0, i, 0)),
            pl.BlockSpec(Wv.shape, lambda i: (0, 0)),
            pl.BlockSpec(Wg.shape, lambda i: (0, 0)),
            pl.BlockSpec((1, DU), lambda i: (0, 0)),
            pl.BlockSpec((1, D), lambda i: (0, 0)),
            pl.BlockSpec((1, DU), lambda i: (0, 0)),
        ],
        out_specs=[
            pl.BlockSpec((BN, D), lambda i: (i, 0)),
            pl.BlockSpec((1, DU), lambda i: (0, 0)),
        ],
        out_shape=[
            jax.ShapeDtypeStruct((N, D), f32),
            jax.ShapeDtypeStruct((1, DU), f32),
        ],
        scratch_shapes=[
            pltpu.VMEM((1, D), f32),
            pltpu.VMEM((1, DE), f32),
        ],
    )(x, p_in, p_out, Wv, Wg, u2, bv2, bg2)

    return x_new, eT.T, u_new.reshape(DU)


# speedup vs baseline: 13.9382x; 1.0545x over previous
import functools

import jax
import jax.numpy as jnp
from jax import lax
from jax.experimental import pallas as pl
from jax.experimental.pallas import tpu as pltpu
from jax.experimental.pallas import tpu_sc as plsc


def _tc_nodes_body(x8_ref, W8s_ref, W8d_ref, xs_ref, xd_ref):
    f32 = jnp.float32
    x8 = x8_ref[...]
    xs_ref[...] = jnp.dot(x8, W8s_ref[...], preferred_element_type=f32)
    xd_ref[...] = jnp.dot(x8, W8d_ref[...], preferred_element_type=f32)


def _tc_edges_body(eaT_ref, We_ref, u_ref, be_ref, tT_ref, *, D, DE):
    f32 = jnp.float32
    ident = (jax.lax.broadcasted_iota(jnp.int32, (DE, DE), 0)
             == jax.lax.broadcasted_iota(jnp.int32, (DE, DE), 1)).astype(f32)
    ceT = lax.dot_general(We_ref[DE + 2 * D:, :], u_ref[...],
                          (((0,), (1,)), ((), ())),
                          preferred_element_type=f32)
    beT = lax.dot_general(ident, be_ref[...],
                          (((1,), (1,)), ((), ())),
                          preferred_element_type=f32)
    tT_ref[...] = lax.dot_general(
        We_ref[:DE, :], eaT_ref[...],
        (((0,), (0,)), ((), ())), preferred_element_type=f32) + ceT + beT


def _tc_node_body(x_ref, pin_ref, pout_ref, Wv_ref, Wg_ref, u_ref, bv_ref,
                  bg_ref, xn_ref, un_ref, accx_ref, acce_ref,
                  *, D, DE, DU, N, E, NBLK):
    f32 = jnp.float32
    i = pl.program_id(0)

    @pl.when(i == 0)
    def _():
        accx_ref[...] = jnp.zeros_like(accx_ref)
        acce_ref[...] = jnp.zeros_like(acce_ref)

    aggin = pin_ref[0] + pin_ref[1]
    aggout = pout_ref[0] + pout_ref[1]
    cv = jnp.dot(u_ref[...], Wv_ref[D + 2 * DE:, :],
                 preferred_element_type=f32) + bv_ref[...]
    xn = (jnp.dot(x_ref[...], Wv_ref[:D, :], preferred_element_type=f32)
          + jnp.dot(aggin, Wv_ref[D:D + DE, :], preferred_element_type=f32)
          + jnp.dot(aggout, Wv_ref[D + DE:D + 2 * DE, :],
                    preferred_element_type=f32)
          + cv)
    xn = jnp.maximum(xn, 0.0)
    xn_ref[...] = xn
    accx_ref[...] += jnp.sum(xn, axis=0, keepdims=True)
    acce_ref[...] += jnp.sum(aggin, axis=0, keepdims=True)

    @pl.when(i == NBLK - 1)
    def _():
        gx = jnp.dot(accx_ref[...] * (1.0 / N), Wg_ref[:D, :],
                     preferred_element_type=f32)
        ge = jnp.dot(acce_ref[...] * (1.0 / E), Wg_ref[D:D + DE, :],
                     preferred_element_type=f32)
        gu = jnp.dot(u_ref[...], Wg_ref[D + DE:, :],
                     preferred_element_type=f32)
        un_ref[...] = jnp.maximum(gx + ge + gu + bg_ref[...], 0.0)


def _make_sc_edge(N, E, DE, NC, NS, C, W):
    NW = NC * NS
    EPW = E // NW
    CPW = W // C
    NWIN = EPW // W
    NPT = (N // NS) // 8 * 8
    NPT_LAST = N - NPT * (NS - 1)
    mesh = plsc.VectorSubcoreMesh(core_axis_name="c", subcore_axis_name="s")
    f32 = jnp.float32

    @functools.partial(
        pl.kernel,
        out_type=[
            jax.ShapeDtypeStruct((DE, E), f32),
            jax.ShapeDtypeStruct((NC, N, DE), f32),
            jax.ShapeDtypeStruct((NC, N, DE), f32),
        ],
        mesh=mesh,
        scratch_types=[
            pltpu.VMEM((EPW,), jnp.int32),
            pltpu.VMEM((EPW,), jnp.int32),
            pltpu.VMEM((C, DE), f32),
            pltpu.VMEM((C, DE), f32),
            pltpu.VMEM((C, DE), f32),
            pltpu.VMEM((C, DE), f32),
            pltpu.VMEM((C, DE), f32),
            pltpu.VMEM((C, DE), f32),
            pltpu.VMEM((DE * W,), f32),
            pltpu.VMEM((DE * W,), f32),
            pltpu.VMEM((NPT_LAST, DE), f32),
            pltpu.VMEM_SHARED((N, DE), f32),
            pltpu.VMEM_SHARED((N, DE), f32),
            pltpu.SemaphoreType.DMA,
            pltpu.SemaphoreType.DMA,
            pltpu.SemaphoreType.DMA,
            pltpu.SemaphoreType.DMA,
            pltpu.SemaphoreType.DMA,
            pltpu.SemaphoreType.DMA,
            pltpu.SemaphoreType.DMA,
        ],
        compiler_params=pltpu.CompilerParams(use_tc_tiling_on_sc=False,
                                             needs_layout_passes=False),
    )
    def sc_edge(xs_hbm, xd_hbm, tT_hbm, src_hbm, dst_hbm,
                eT_hbm, pin_hbm, pout_hbm,
                src_all, dst_all, xsb0, xsb1, xdb0, xdb1,
                eb0, eb1, tbT, ebT, zb, agg_in, agg_out,
                semi, semg0, semg1, semT, semF, semw0, semw1):
        cid = lax.axis_index("c")
        sid = lax.axis_index("s")
        wid = sid * NC + cid
        ebase = wid * EPW

        xsb = (xsb0, xsb1)
        xdb = (xdb0, xdb1)
        eb = (eb0, eb1)
        semg = (semg0, semg1)
        ivW = lax.iota(jnp.int32, 16) * W

        pltpu.async_copy(src_hbm.at[pl.ds(ebase, EPW)], src_all, semi)
        pltpu.async_copy(dst_hbm.at[pl.ds(ebase, EPW)], dst_all, semi)

        zrow = jnp.zeros((16,), f32)

        def zfill(i, _):
            zb[i] = zrow
            return 0

        lax.fori_loop(0, NPT_LAST, zfill, 0, unroll=8)
        nbase = sid * NPT

        @pl.when(sid < NS - 1)
        def _():
            pltpu.sync_copy(zb.at[pl.ds(0, NPT)], agg_in.at[pl.ds(nbase, NPT)])
            pltpu.sync_copy(zb.at[pl.ds(0, NPT)], agg_out.at[pl.ds(nbase, NPT)])

        @pl.when(sid == NS - 1)
        def _():
            pltpu.sync_copy(zb, agg_in.at[pl.ds(nbase, NPT_LAST)])
            pltpu.sync_copy(zb, agg_out.at[pl.ds(nbase, NPT_LAST)])

        pltpu.make_async_copy(src_hbm.at[pl.ds(ebase, EPW)], src_all,
                              semi).wait()
        pltpu.make_async_copy(dst_hbm.at[pl.ds(ebase, EPW)], dst_all,
                              semi).wait()
        plsc.subcore_barrier()

        def issue_gathers(j, s):
            off = pl.multiple_of(j * C, 8)
            pltpu.async_copy(xs_hbm.at[src_all.at[pl.ds(off, C)]],
                             xsb[s], semg[s])
            pltpu.async_copy(xd_hbm.at[dst_all.at[pl.ds(off, C)]],
                             xdb[s], semg[s])

        def drain_gathers(s):
            pltpu.make_async_copy(xs_hbm.at[pl.ds(0, C)], xsb[s],
                                  semg[s]).wait()
            pltpu.make_async_copy(xd_hbm.at[pl.ds(0, C)], xdb[s],
                                  semg[s]).wait()

        def compute(c, s):
            @plsc.parallel_loop(0, C // 8, unroll=2)
            def _(q):
                for a in range(8):
                    i = q * 8 + a
                    idxv = ivW + (c * C + i)
                    tv = plsc.load_gather(tbT, [idxv])
                    v = jnp.maximum(tv + xsb[s][i] + xdb[s][i], 0.0)
                    eb[s][i] = v
                    plsc.store_scatter(ebT, [idxv], v)

        semw = (semw0, semw1)

        def issue_writes(j, s):
            off = pl.multiple_of(j * C, 8)
            pltpu.async_copy(eb[s], agg_in.at[dst_all.at[pl.ds(off, C)]],
                             semw[s], add=True)
            pltpu.async_copy(eb[s], agg_out.at[src_all.at[pl.ds(off, C)]],
                             semw[s], add=True)

        def drain_writes(s):
            pltpu.make_async_copy(eb[s], agg_in.at[dst_all.at[pl.ds(0, C)]],
                                  semw[s]).wait()
            pltpu.make_async_copy(eb[s], agg_out.at[src_all.at[pl.ds(0, C)]],
                                  semw[s]).wait()

        for w in range(NWIN):
            wbase = ebase + w * W
            for f in range(DE):
                pltpu.async_copy(tT_hbm.at[f, pl.ds(wbase, W)],
                                 tbT.at[pl.ds(f * W, W)], semT)
            if w > 0:
                pwbase = ebase + (w - 1) * W
                for f in range(DE):
                    pltpu.make_async_copy(ebT.at[pl.ds(f * W, W)],
                                          eT_hbm.at[f, pl.ds(pwbase, W)],
                                          semF).wait()
            for f in range(DE):
                pltpu.make_async_copy(tT_hbm.at[f, pl.ds(wbase, W)],
                                      tbT.at[pl.ds(f * W, W)], semT).wait()

            gbase = w * CPW

            first_window = (w == 0)

            def stage(c, s):
                @pl.when(c + 1 < CPW)
                def _():
                    issue_gathers(gbase + c + 1, (s + 1) % 2)

                @pl.when(c < CPW)
                def _():
                    if first_window:
                        @pl.when(c >= 2)
                        def _():
                            drain_writes(s)
                    else:
                        drain_writes(s)
                    drain_gathers(s)
                    compute(c, s)
                    issue_writes(gbase + c, s)

            issue_gathers(gbase, 0)

            def body2(cc, _):
                stage(2 * cc, 0)
                stage(2 * cc + 1, 1)
                return 0

            lax.fori_loop(0, (CPW + 1) // 2, body2, 0)

            for f in range(DE):
                pltpu.async_copy(ebT.at[pl.ds(f * W, W)],
                                 eT_hbm.at[f, pl.ds(wbase, W)], semF)

        lwbase = ebase + (NWIN - 1) * W
        for f in range(DE):
            pltpu.make_async_copy(ebT.at[pl.ds(f * W, W)],
                                  eT_hbm.at[f, pl.ds(lwbase, W)], semF).wait()
        drain_writes(0)
        drain_writes(1)

        plsc.subcore_barrier()

        @pl.when(sid < NS - 1)
        def _():
            pltpu.sync_copy(agg_in.at[pl.ds(nbase, NPT)],
                            pin_hbm.at[cid, pl.ds(nbase, NPT)])
            pltpu.sync_copy(agg_out.at[pl.ds(nbase, NPT)],
                            pout_hbm.at[cid, pl.ds(nbase, NPT)])

        @pl.when(sid == NS - 1)
        def _():
            pltpu.sync_copy(agg_in.at[pl.ds(nbase, NPT_LAST)],
                            pin_hbm.at[cid, pl.ds(nbase, NPT_LAST)])
            pltpu.sync_copy(agg_out.at[pl.ds(nbase, NPT_LAST)],
                            pout_hbm.at[cid, pl.ds(nbase, NPT_LAST)])

    return sc_edge


def kernel(x, edge_attr, edge_index, u, We, be, Wv, bv, Wg, bg):
    N, D = x.shape
    E, DE = edge_attr.shape
    DU = u.shape[0]
    f32 = jnp.float32

    src = edge_index[0]
    dst = edge_index[1]
    u2 = u.reshape(1, DU)
    be2 = be.reshape(1, DE)
    bv2 = bv.reshape(1, D)
    bg2 = bg.reshape(1, DU)

    NBLK = 10
    BN = N // NBLK
    BE = 6400
    EBLK = E // BE

    x8 = x.reshape(N // 8, 8 * D)
    eye8 = jnp.eye(8, dtype=f32)
    W8s = jnp.kron(eye8, We[DE:DE + D, :])
    W8d = jnp.kron(eye8, We[DE + D:DE + 2 * D, :])
    xsp, xdp = pl.pallas_call(
        _tc_nodes_body,
        grid=(1,),
        in_specs=[
            pl.BlockSpec((N // 8, 8 * D), lambda i: (0, 0)),
            pl.BlockSpec((8 * D, 8 * DE), lambda i: (0, 0)),
            pl.BlockSpec((8 * D, 8 * DE), lambda i: (0, 0)),
        ],
        out_specs=[
            pl.BlockSpec((N // 8, 8 * DE), lambda i: (0, 0)),
            pl.BlockSpec((N // 8, 8 * DE), lambda i: (0, 0)),
        ],
        out_shape=[
            jax.ShapeDtypeStruct((N // 8, 8 * DE), f32),
            jax.ShapeDtypeStruct((N // 8, 8 * DE), f32),
        ],
    )(x8, W8s, W8d)
    xs = xsp.reshape(N, DE)
    xd = xdp.reshape(N, DE)

    eaT = edge_attr.T
    tT = pl.pallas_call(
        functools.partial(_tc_edges_body, D=D, DE=DE),
        grid=(EBLK,),
        in_specs=[
            pl.BlockSpec((DE, BE), lambda i: (0, i)),
            pl.BlockSpec(We.shape, lambda i: (0, 0)),
            pl.BlockSpec((1, DU), lambda i: (0, 0)),
            pl.BlockSpec((1, DE), lambda i: (0, 0)),
        ],
        out_specs=pl.BlockSpec((DE, BE), lambda i: (0, i)),
        out_shape=jax.ShapeDtypeStruct((DE, E), f32),
    )(eaT, We, u2, be2)

    info = plsc.get_sparse_core_info()
    NC, NS = info.num_cores, info.num_subcores
    sc_edge = _make_sc_edge(N, E, DE, NC, NS, C=80, W=2000)
    eT, p_in, p_out = sc_edge(xs, xd, tT, src, dst)

    x_new, u_new = pl.pallas_call(
        functools.partial(_tc_node_body, D=D, DE=DE, DU=DU, N=N, E=E,
                          NBLK=NBLK),
        grid=(NBLK,),
        in_specs=[
            pl.BlockSpec((BN, D), lambda i: (i, 0)),
            pl.BlockSpec((NC, BN, DE), lambda i: (0, i, 0)),
            pl.BlockSpec((NC, BN, DE), lambda i: (0, i, 0)),
            pl.BlockSpec(Wv.shape, lambda i: (0, 0)),
            pl.BlockSpec(Wg.shape, lambda i: (0, 0)),
            pl.BlockSpec((1, DU), lambda i: (0, 0)),
            pl.BlockSpec((1, D), lambda i: (0, 0)),
            pl.BlockSpec((1, DU), lambda i: (0, 0)),
        ],
        out_specs=[
            pl.BlockSpec((BN, D), lambda i: (i, 0)),
            pl.BlockSpec((1, DU), lambda i: (0, 0)),
        ],
        out_shape=[
            jax.ShapeDtypeStruct((N, D), f32),
            jax.ShapeDtypeStruct((1, DU), f32),
        ],
        scratch_shapes=[
            pltpu.VMEM((1, D), f32),
            pltpu.VMEM((1, DE), f32),
        ],
    )(x, p_in, p_out, Wv, Wg, u2, bv2, bg2)

    return x_new, eT.T, u_new.reshape(DU)
